# baseline (speedup 1.0000x reference)
import jax
import jax.numpy as jnp
from jax.experimental import pallas as pl


def _add_pe_block(x_ref, pe_ref, o_ref):
    o_ref[...] = x_ref[...] + pe_ref[...][None, :, :]


def kernel(x, pe, position_ids):
    batch, seq_len, d_model = x.shape
    blk = 1024
    grid = (seq_len // blk, batch)
    return pl.pallas_call(
        _add_pe_block,
        grid=grid,
        in_specs=[
            pl.BlockSpec((1, blk, d_model), lambda i, j: (j, i, 0)),
            pl.BlockSpec((blk, d_model), lambda i, j: (i, 0)),
        ],
        out_specs=pl.BlockSpec((1, blk, d_model), lambda i, j: (j, i, 0)),
        out_shape=jax.ShapeDtypeStruct(x.shape, x.dtype),
    )(x, pe[:seq_len])
